# Initial kernel scaffold; baseline (speedup 1.0000x reference)
import jax
import jax.numpy as jnp
from jax.experimental import pallas as pl


def kernel(node_features, vn_features, rigids_rot, rigids_trans, sidechain, edge_features, params, edge_index, batch, res_mask, noising_mask):
    raise NotImplementedError("write your pallas kernel here")


# speedup vs baseline: 8.4551x; 8.4551x over previous
import functools
import math

import jax
import jax.numpy as jnp
import numpy as np
from jax.experimental import pallas as pl

_H = 8
_PQK = 4
_PV = 8
_CH = 16


def _blk(n, target):
    best = None
    for d in range(8, min(n, target) + 1, 8):
        if n % d == 0:
            best = d
    return best if best is not None else n


def _full(shape):
    return pl.BlockSpec(shape, lambda i: tuple(0 for _ in shape))


def _rows(bs, width):
    return pl.BlockSpec((bs, width), lambda i: (i, 0))


def _proj_body(nf, sc, rt, wf1, wf2, fb, wq, bq, wk, bk, wv, bv, wqp, bqp,
               wkp, bkp, wvp, bvp, q_o, k_o, v_o, qpg_o, kpg_o, vpg_o):
    x = jnp.dot(nf[...], wf1[...], preferred_element_type=jnp.float32)
    x = x + jnp.dot(sc[...], wf2[...], preferred_element_type=jnp.float32)
    x = x + fb[...]
    q_o[...] = jnp.dot(x, wq[...], preferred_element_type=jnp.float32) + bq[...]
    k_o[...] = jnp.dot(x, wk[...], preferred_element_type=jnp.float32) + bk[...]
    v_o[...] = jnp.dot(x, wv[...], preferred_element_type=jnp.float32) + bv[...]
    r = rt[...]

    def to_global(w, b, npts):
        p = jnp.dot(x, w[...], preferred_element_type=jnp.float32) + b[...]
        px, py, pz = p[:, :npts], p[:, npts:2 * npts], p[:, 2 * npts:]
        outs = []
        for i in range(3):
            gi = (r[:, 3 * i:3 * i + 1] * px + r[:, 3 * i + 1:3 * i + 2] * py
                  + r[:, 3 * i + 2:3 * i + 3] * pz + r[:, 9 + i:10 + i])
            outs.append(gi)
        return jnp.concatenate(outs, axis=1)

    qpg_o[...] = to_global(wqp, bqp, _H * _PQK)
    kpg_o[...] = to_global(wkp, bkp, _H * _PQK)
    vpg_o[...] = to_global(wvp, bvp, _H * _PV)


def _logit_body(qd, ks, qpd, kps, z, wbz, bbz, gam, s16, s4, a_o):
    c1 = 1.0 / math.sqrt(3.0 * _CH)
    c2 = 1.0 / math.sqrt(3.0)
    c3 = 0.5 * math.sqrt(1.0 / (3.0 * _PQK * 4.5))
    prod = qd[...] * ks[...]
    dot = jnp.dot(prod, s16[...], preferred_element_type=jnp.float32)
    diff = qpd[...] - kps[...]
    d2 = diff * diff
    npts = _H * _PQK
    d2s = (d2[:, :npts] + d2[:, npts:2 * npts] + d2[:, 2 * npts:])
    d2h = jnp.dot(d2s, s4[...], preferred_element_type=jnp.float32)
    be = jnp.dot(z[...], wbz[...], preferred_element_type=jnp.float32) + bbz[...]
    a_o[...] = dot * c1 + be * c2 - (c3 * gam[...]) * d2h


def _msg_body(attn, vs, z, vps, wo, wpair, e16, e192, ms_o, wvp_o):
    aw = attn[...]
    B = aw.shape[0]
    rep16 = jnp.dot(aw, e16[...], preferred_element_type=jnp.float32)
    wv = rep16 * vs[...]
    mpart = jnp.dot(wv, wo[...], preferred_element_type=jnp.float32)
    y = jnp.dot(z[...], wpair[...], preferred_element_type=jnp.float32)
    acc = jnp.zeros((B, 128), jnp.float32)
    for h in range(_H):
        acc = acc + aw[:, h:h + 1] * y[:, h * 128:(h + 1) * 128]
    ms_o[...] = mpart + acc
    rep192 = jnp.dot(aw, e192[...], preferred_element_type=jnp.float32)
    wvp_o[...] = rep192 * vps[...]


def _node1_body(nf, aggs, aggp, rt, wptx, wpty, wptz, wnorm, ob, lg, lb,
                wvnk, wvnv, n1_o, kn_o, vv_o):
    r = rt[...]
    g = aggp[...]
    npts = _H * _PV
    gx, gy, gz = g[:, :npts], g[:, npts:2 * npts], g[:, 2 * npts:]
    gx = gx - r[:, 9:10]
    gy = gy - r[:, 10:11]
    gz = gz - r[:, 11:12]
    opt = []
    for i in range(3):
        opt.append(r[:, i:i + 1] * gx + r[:, 3 + i:4 + i] * gy
                   + r[:, 6 + i:7 + i] * gz)
    nrm = jnp.sqrt(opt[0] * opt[0] + opt[1] * opt[1] + opt[2] * opt[2] + 1e-8)
    s = aggs[...] + ob[...]
    s = s + jnp.dot(opt[0], wptx[...], preferred_element_type=jnp.float32)
    s = s + jnp.dot(opt[1], wpty[...], preferred_element_type=jnp.float32)
    s = s + jnp.dot(opt[2], wptz[...], preferred_element_type=jnp.float32)
    s = s + jnp.dot(nrm, wnorm[...], preferred_element_type=jnp.float32)
    y = nf[...] + s
    mu = jnp.mean(y, axis=-1, keepdims=True)
    var = jnp.mean((y - mu) ** 2, axis=-1, keepdims=True)
    n1 = lg[...] * (y - mu) / jnp.sqrt(var + 1e-5) + lb[...]
    n1_o[...] = n1
    kn_o[...] = jnp.dot(n1, wvnk[...], preferred_element_type=jnp.float32)
    vv_o[...] = jnp.dot(n1, wvnv[...], preferred_element_type=jnp.float32)


def _vn_body(vn, pooled, wq_, wo_, bo_, wn_, bn_, qv_o, vnn_o, vnnt_o):
    v = vn[...]
    qv_o[...] = jnp.dot(v, wq_[...], preferred_element_type=jnp.float32)
    vnew = v + jnp.dot(pooled[...], wo_[...],
                       preferred_element_type=jnp.float32) + bo_[...]
    vnn_o[...] = vnew
    vnnt_o[...] = jnp.dot(vnew, wn_[...],
                          preferred_element_type=jnp.float32) + bn_[...]


def _final_body(n1, vt, rt, sc, w1, b1, w2, b2, w3, b3, tg, tb, wbb, bbb,
                wsc, bsc, wen, ben, n3_o, rto_o, side_o, hn_o):
    node2 = n1[...] + vt[...]
    h = jnp.maximum(jnp.dot(node2, w1[...],
                            preferred_element_type=jnp.float32) + b1[...], 0.0)
    h = jnp.maximum(jnp.dot(h, w2[...],
                            preferred_element_type=jnp.float32) + b2[...], 0.0)
    h = jnp.dot(h, w3[...], preferred_element_type=jnp.float32) + b3[...]
    y = node2 + h
    mu = jnp.mean(y, axis=-1, keepdims=True)
    var = jnp.mean((y - mu) ** 2, axis=-1, keepdims=True)
    n3 = tg[...] * (y - mu) / jnp.sqrt(var + 1e-5) + tb[...]
    n3_o[...] = n3
    r = rt[...]
    nm = r[:, 12:13]
    u = nm * (jnp.dot(n3, wbb[...], preferred_element_type=jnp.float32)
              + bbb[...])
    qx, qy, qz = u[:, 0:1], u[:, 1:2], u[:, 2:3]
    inv = 1.0 / jnp.sqrt(1.0 + qx * qx + qy * qy + qz * qz + 1e-12)
    w_, x_, y_, z_ = inv, qx * inv, qy * inv, qz * inv
    Ru = [1 - 2 * (y_ * y_ + z_ * z_), 2 * (x_ * y_ - z_ * w_), 2 * (x_ * z_ + y_ * w_),
          2 * (x_ * y_ + z_ * w_), 1 - 2 * (x_ * x_ + z_ * z_), 2 * (y_ * z_ - x_ * w_),
          2 * (x_ * z_ - y_ * w_), 2 * (y_ * z_ + x_ * w_), 1 - 2 * (x_ * x_ + y_ * y_)]
    pieces = []
    for i in range(3):
        for k in range(3):
            acc = (r[:, 3 * i:3 * i + 1] * Ru[0 + k] + r[:, 3 * i + 1:3 * i + 2] * Ru[3 + k]
                   + r[:, 3 * i + 2:3 * i + 3] * Ru[6 + k])
            pieces.append(acc)
    for i in range(3):
        acc = (r[:, 9 + i:10 + i] + r[:, 3 * i:3 * i + 1] * u[:, 3:4]
               + r[:, 3 * i + 1:3 * i + 2] * u[:, 4:5]
               + r[:, 3 * i + 2:3 * i + 3] * u[:, 5:6])
        pieces.append(acc)
    B = n3.shape[0]
    lane = jax.lax.broadcasted_iota(jnp.int32, (B, 16), 1)
    rto = jnp.zeros((B, 16), jnp.float32)
    for c, piece in enumerate(pieces):
        rto = rto + jnp.where(lane == c, piece, 0.0)
    rto_o[...] = rto
    side_o[...] = sc[...] + nm * (
        jnp.dot(n3, wsc[...], preferred_element_type=jnp.float32) + bsc[...])
    hn_o[...] = jnp.dot(n3, wen[...], preferred_element_type=jnp.float32) + ben[...]


def _etrans_body(hs, hd, z, w1s, w1d, w1z, b1, w2, b2, eg, eb, e_o):
    t = (jnp.dot(hs[...], w1s[...], preferred_element_type=jnp.float32)
         + jnp.dot(hd[...], w1d[...], preferred_element_type=jnp.float32)
         + jnp.dot(z[...], w1z[...], preferred_element_type=jnp.float32)
         + b1[...])
    t = jnp.maximum(t, 0.0)
    e = jnp.dot(t, w2[...], preferred_element_type=jnp.float32) + b2[...]
    mu = jnp.mean(e, axis=-1, keepdims=True)
    var = jnp.mean((e - mu) ** 2, axis=-1, keepdims=True)
    e_o[...] = eg[...] * (e - mu) / jnp.sqrt(var + 1e-5) + eb[...]


def _call(body, grid, in_specs, out_specs, out_shapes, args):
    return pl.pallas_call(
        body, grid=grid, in_specs=in_specs, out_specs=out_specs,
        out_shape=out_shapes)(*args)


@jax.jit
def kernel(node_features, vn_features, rigids_rot, rigids_trans, sidechain,
           edge_features, params, edge_index, batch, res_mask, noising_mask):
    p = params
    N, C_S = node_features.shape
    E, C_Z = edge_features.shape
    G = vn_features.shape[0]
    C_LAT = sidechain.shape[1]
    src, dst = edge_index[0], edge_index[1]
    f32 = jnp.float32

    def pts_perm(npts_in, sel):
        idx = []
        for j in range(3):
            for h in range(_H):
                for q in sel:
                    idx.append(h * npts_in * 3 + q * 3 + j)
        return np.array(idx, np.int32)

    qp_perm = pts_perm(_PQK, range(_PQK))
    kp_perm = pts_perm(_PQK + _PV, range(_PQK))
    vp_perm = pts_perm(_PQK + _PV, range(_PQK, _PQK + _PV))
    wqp = p['qp_W'][:, qp_perm]
    bqp = p['qp_b'][qp_perm][None]
    wkp = p['kvp_W'][:, kp_perm]
    bkp = p['kvp_b'][kp_perm][None]
    wvp = p['kvp_W'][:, vp_perm]
    bvp = p['kvp_b'][vp_perm][None]
    kcols = np.array([h * 2 * _CH + c for h in range(_H) for c in range(_CH)])
    vcols = kcols + _CH
    wk = p['kv_W'][:, kcols]
    bk = p['kv_b'][kcols][None]
    wv = p['kv_W'][:, vcols]
    bv = p['kv_b'][vcols][None]
    npv = _H * _PV
    Wo = p['out_W'][:C_S]
    pt_rows = lambda i: np.array(
        [C_S + h * _PV * 3 + q * 3 + i for h in range(_H) for q in range(_PV)])
    Wptx = p['out_W'][pt_rows(0)]
    Wpty = p['out_W'][pt_rows(1)]
    Wptz = p['out_W'][pt_rows(2)]
    Wnorm = p['out_W'][C_S + 3 * npv: C_S + 4 * npv]
    Wpair = (p['out_W'][C_S + 4 * npv:]
             .reshape(_H, C_Z, C_S).transpose(1, 0, 2).reshape(C_Z, _H * C_S))

    rt = jnp.concatenate(
        [rigids_rot.reshape(N, 9), rigids_trans,
         noising_mask[:, None].astype(f32), jnp.zeros((N, 3), f32)], axis=1)

    bn = _blk(N, 1000)
    be = _blk(E, 1600)
    bec = _blk(E, 800)

    q, k, v, qpg, kpg, vpg = _call(
        _proj_body, (N // bn,),
        [_rows(bn, C_S), _rows(bn, C_LAT), _rows(bn, 16),
         _full((C_S, C_S)), _full((C_LAT, C_S)), _full((1, C_S)),
         _full((C_S, C_S)), _full((1, C_S)), _full((C_S, C_S)), _full((1, C_S)),
         _full((C_S, C_S)), _full((1, C_S)),
         _full((C_S, 96)), _full((1, 96)), _full((C_S, 96)), _full((1, 96)),
         _full((C_S, 192)), _full((1, 192))],
        [_rows(bn, C_S)] * 3 + [_rows(bn, 96), _rows(bn, 96), _rows(bn, 192)],
        [jax.ShapeDtypeStruct((N, C_S), f32)] * 3
        + [jax.ShapeDtypeStruct((N, 96), f32)] * 2
        + [jax.ShapeDtypeStruct((N, 192), f32)],
        (node_features, sidechain, rt,
         p['fuse_W'][:C_S], p['fuse_W'][C_S:], p['fuse_b'][None],
         p['q_W'], p['q_b'][None], wk, bk, wv, bv,
         wqp, bqp, wkp, bkp, wvp, bvp))

    gamma = jax.nn.softplus(p['head_w'])[None]

    def onehot(rows, f):
        mat = np.zeros((rows, _H), np.float32)
        for r_ in range(rows):
            mat[r_, f(r_)] = 1.0
        return jnp.asarray(mat)

    s16 = onehot(C_S, lambda r_: r_ // _CH)
    s4 = onehot(_H * _PQK, lambda r_: r_ // _PQK)
    e16 = onehot(C_S, lambda r_: r_ // _CH).T
    e192 = onehot(192, lambda r_: (r_ % 64) // _PV).T

    (a,) = _call(
        _logit_body, (E // be,),
        [_rows(be, C_S), _rows(be, C_S), _rows(be, 96), _rows(be, 96),
         _rows(be, C_Z), _full((C_Z, _H)), _full((1, _H)), _full((1, _H)),
         _full((C_S, _H)), _full((_H * _PQK, _H))],
        [_rows(be, _H)],
        [jax.ShapeDtypeStruct((E, _H), f32)],
        (jnp.take(q, dst, axis=0), jnp.take(k, src, axis=0),
         jnp.take(qpg, dst, axis=0), jnp.take(kpg, src, axis=0),
         edge_features, p['bz_W'], p['bz_b'][None], gamma, s16, s4))

    m = jax.ops.segment_max(a, dst, num_segments=N)
    m = jnp.where(jnp.isfinite(m), m, 0.0)
    ex = jnp.exp(a - m[dst])
    den = jax.ops.segment_sum(ex, dst, num_segments=N)
    attn = ex / (den[dst] + 1e-9)

    msg_s, wvp_e = _call(
        _msg_body, (E // bec,),
        [_rows(bec, _H), _rows(bec, C_S), _rows(bec, C_Z), _rows(bec, 192),
         _full((C_S, C_S)), _full((C_Z, _H * C_S)), _full((_H, C_S)),
         _full((_H, 192))],
        [_rows(bec, C_S), _rows(bec, 192)],
        [jax.ShapeDtypeStruct((E, C_S), f32),
         jax.ShapeDtypeStruct((E, 192), f32)],
        (attn, jnp.take(v, src, axis=0), edge_features,
         jnp.take(vpg, src, axis=0), Wo, Wpair, e16, e192))

    agg_s = jax.ops.segment_sum(msg_s, dst, num_segments=N)
    agg_p = jax.ops.segment_sum(wvp_e, dst, num_segments=N)

    node1, kn, vv = _call(
        _node1_body, (N // bn,),
        [_rows(bn, C_S), _rows(bn, C_S), _rows(bn, 192), _rows(bn, 16),
         _full((npv, C_S)), _full((npv, C_S)), _full((npv, C_S)),
         _full((npv, C_S)), _full((1, C_S)), _full((1, C_S)), _full((1, C_S)),
         _full((C_S, C_S)), _full((C_S, C_S))],
        [_rows(bn, C_S)] * 3,
        [jax.ShapeDtypeStruct((N, C_S), f32)] * 3,
        (node_features, agg_s, agg_p, rt, Wptx, Wpty, Wptz, Wnorm,
         p['out_b'][None], p['ln1_g'][None], p['ln1_b'][None],
         p['vnk_W'], p['vnv_W']))

    Gp = ((G + 7) // 8) * 8
    vn_pad = jnp.pad(vn_features, ((0, Gp - G), (0, 0)))
    dh = C_S // _H

    qv_p, _, _ = _call(
        _vn_body, (1,),
        [_full((Gp, C_S)), _full((Gp, C_S))] + [_full((C_S, C_S)),
         _full((C_S, C_S)), _full((1, C_S)), _full((C_S, C_S)), _full((1, C_S))],
        [_full((Gp, C_S))] * 3,
        [jax.ShapeDtypeStruct((Gp, C_S), f32)] * 3,
        (vn_pad, jnp.zeros((Gp, C_S), f32), p['vnq_W'], p['vno_W'],
         p['vno_b'][None], p['vnn_W'], p['vnn_b'][None]))
    qv = qv_p[:G]

    lg = jnp.sum((qv[batch] * kn).reshape(N, _H, dh), -1) / math.sqrt(dh)
    m2 = jax.ops.segment_max(lg, batch, num_segments=G)
    m2 = jnp.where(jnp.isfinite(m2), m2, 0.0)
    ex2 = jnp.exp(lg - m2[batch])
    den2 = jax.ops.segment_sum(ex2, batch, num_segments=G)
    at2 = ex2 / (den2[batch] + 1e-9)
    wvv = jnp.repeat(at2, dh, axis=1) * vv
    pooled = jax.ops.segment_sum(wvv, batch, num_segments=G)

    _, vn_new_p, vnnt_p = _call(
        _vn_body, (1,),
        [_full((Gp, C_S)), _full((Gp, C_S))] + [_full((C_S, C_S)),
         _full((C_S, C_S)), _full((1, C_S)), _full((C_S, C_S)), _full((1, C_S))],
        [_full((Gp, C_S))] * 3,
        [jax.ShapeDtypeStruct((Gp, C_S), f32)] * 3,
        (vn_pad, jnp.pad(pooled, ((0, Gp - G), (0, 0))), p['vnq_W'],
         p['vno_W'], p['vno_b'][None], p['vnn_W'], p['vnn_b'][None]))
    vn_new = vn_new_p[:G]
    vt = vnnt_p[:G][batch]

    bb_Wp = jnp.pad(p['bb_W'], ((0, 0), (0, 2)))
    bb_bp = jnp.pad(p['bb_b'], (0, 2))[None]
    node3, rto, side_new, hn = _call(
        _final_body, (N // bn,),
        [_rows(bn, C_S), _rows(bn, C_S), _rows(bn, 16), _rows(bn, C_LAT),
         _full((C_S, C_S)), _full((1, C_S)), _full((C_S, C_S)), _full((1, C_S)),
         _full((C_S, C_S)), _full((1, C_S)), _full((1, C_S)), _full((1, C_S)),
         _full((C_S, 8)), _full((1, 8)), _full((C_S, C_LAT)), _full((1, C_LAT)),
         _full((C_S, 64)), _full((1, 64))],
        [_rows(bn, C_S), _rows(bn, 16), _rows(bn, C_LAT), _rows(bn, 64)],
        [jax.ShapeDtypeStruct((N, C_S), f32),
         jax.ShapeDtypeStruct((N, 16), f32),
         jax.ShapeDtypeStruct((N, C_LAT), f32),
         jax.ShapeDtypeStruct((N, 64), f32)],
        (node1, vt, rt, sidechain,
         p['t1_W'], p['t1_b'][None], p['t2_W'], p['t2_b'][None],
         p['t3_W'], p['t3_b'][None], p['tln_g'][None], p['tln_b'][None],
         bb_Wp, bb_bp, p['sc_W'], p['sc_b'][None], p['en_W'], p['en_b'][None]))

    new_R = rto[:, :9].reshape(N, 3, 3)
    new_t = rto[:, 9:12]

    (e,) = _call(
        _etrans_body, (E // be,),
        [_rows(be, 64), _rows(be, 64), _rows(be, C_Z),
         _full((64, C_Z)), _full((64, C_Z)), _full((C_Z, C_Z)), _full((1, C_Z)),
         _full((C_Z, C_Z)), _full((1, C_Z)), _full((1, C_Z)), _full((1, C_Z))],
        [_rows(be, C_Z)],
        [jax.ShapeDtypeStruct((E, C_Z), f32)],
        (jnp.take(hn, src, axis=0), jnp.take(hn, dst, axis=0), edge_features,
         p['e1_W'][:64], p['e1_W'][64:128], p['e1_W'][128:], p['e1_b'][None],
         p['e2_W'], p['e2_b'][None], p['eln_g'][None], p['eln_b'][None]))

    return (node3, vn_new, new_R, new_t, side_new, e)


# speedup vs baseline: 9.8896x; 1.1697x over previous
import functools
import math

import jax
import jax.numpy as jnp
import numpy as np
from jax.experimental import pallas as pl

_H = 8
_PQK = 4
_PV = 8
_CH = 16


def _blk(n, target):
    best = None
    for d in range(8, min(n, target) + 1, 8):
        if n % d == 0:
            best = d
    return best if best is not None else n


def _full(shape):
    return pl.BlockSpec(shape, lambda i: tuple(0 for _ in shape))


def _rows(bs, width):
    return pl.BlockSpec((bs, width), lambda i: (i, 0))


def _proj_body(nf, sc, rt, wf1, wf2, fb, wq, bq, wk, bk, wv, bv, wqp, bqp,
               wkp, bkp, wvp, bvp, qc_o, kc_o, vc_o):
    x = jnp.dot(nf[...], wf1[...], preferred_element_type=jnp.float32)
    x = x + jnp.dot(sc[...], wf2[...], preferred_element_type=jnp.float32)
    x = x + fb[...]
    r = rt[...]

    def to_global(w, b, npts):
        p = jnp.dot(x, w[...], preferred_element_type=jnp.float32) + b[...]
        px, py, pz = p[:, :npts], p[:, npts:2 * npts], p[:, 2 * npts:]
        outs = []
        for i in range(3):
            gi = (r[:, 3 * i:3 * i + 1] * px + r[:, 3 * i + 1:3 * i + 2] * py
                  + r[:, 3 * i + 2:3 * i + 3] * pz + r[:, 9 + i:10 + i])
            outs.append(gi)
        return jnp.concatenate(outs, axis=1)

    qc_o[:, :128] = jnp.dot(x, wq[...], preferred_element_type=jnp.float32) + bq[...]
    qc_o[:, 128:] = to_global(wqp, bqp, _H * _PQK)
    kc_o[:, :128] = jnp.dot(x, wk[...], preferred_element_type=jnp.float32) + bk[...]
    kc_o[:, 128:] = to_global(wkp, bkp, _H * _PQK)
    vc_o[:, :128] = jnp.dot(x, wv[...], preferred_element_type=jnp.float32) + bv[...]
    vc_o[:, 128:] = to_global(wvp, bvp, _H * _PV)


def _logit_body(qc, kc, z, wbz, bbz, gam, s16, s4, a_o):
    c1 = 1.0 / math.sqrt(3.0 * _CH)
    c2 = 1.0 / math.sqrt(3.0)
    c3 = 0.5 * math.sqrt(1.0 / (3.0 * _PQK * 4.5))
    qd, qpd = qc[:, :128], qc[:, 128:]
    ks, kps = kc[:, :128], kc[:, 128:]
    prod = qd * ks
    dot = jnp.dot(prod, s16[...], preferred_element_type=jnp.float32)
    diff = qpd - kps
    d2 = diff * diff
    npts = _H * _PQK
    d2s = (d2[:, :npts] + d2[:, npts:2 * npts] + d2[:, 2 * npts:])
    d2h = jnp.dot(d2s, s4[...], preferred_element_type=jnp.float32)
    be = jnp.dot(z[...], wbz[...], preferred_element_type=jnp.float32) + bbz[...]
    a_o[...] = dot * c1 + be * c2 - (c3 * gam[...]) * d2h


def _msg_body(a_in, mdg, vc, z, wo, wpair, e16, e192, ms_o):
    md = mdg[...]
    ex = jnp.exp(a_in[...] - md[:, :_H])
    aw = ex / (md[:, _H:] + 1e-9)
    B = aw.shape[0]
    rep16 = jnp.dot(aw, e16[...], preferred_element_type=jnp.float32)
    wv = rep16 * vc[:, :128]
    mpart = jnp.dot(wv, wo[...], preferred_element_type=jnp.float32)
    y = jnp.dot(z[...], wpair[...], preferred_element_type=jnp.float32)
    acc = jnp.zeros((B, 128), jnp.float32)
    for h in range(_H):
        acc = acc + aw[:, h:h + 1] * y[:, h * 128:(h + 1) * 128]
    ms_o[:, :128] = mpart + acc
    rep192 = jnp.dot(aw, e192[...], preferred_element_type=jnp.float32)
    ms_o[:, 128:] = rep192 * vc[:, 128:]


def _node1_body(nf, agg, rt, wptx, wpty, wptz, wnorm, ob, lg, lb,
                wvnk, wvnv, n1_o, kn_o, vv_o):
    r = rt[...]
    g = agg[:, 128:]
    npts = _H * _PV
    gx, gy, gz = g[:, :npts], g[:, npts:2 * npts], g[:, 2 * npts:]
    gx = gx - r[:, 9:10]
    gy = gy - r[:, 10:11]
    gz = gz - r[:, 11:12]
    opt = []
    for i in range(3):
        opt.append(r[:, i:i + 1] * gx + r[:, 3 + i:4 + i] * gy
                   + r[:, 6 + i:7 + i] * gz)
    nrm = jnp.sqrt(opt[0] * opt[0] + opt[1] * opt[1] + opt[2] * opt[2] + 1e-8)
    s = agg[:, :128] + ob[...]
    s = s + jnp.dot(opt[0], wptx[...], preferred_element_type=jnp.float32)
    s = s + jnp.dot(opt[1], wpty[...], preferred_element_type=jnp.float32)
    s = s + jnp.dot(opt[2], wptz[...], preferred_element_type=jnp.float32)
    s = s + jnp.dot(nrm, wnorm[...], preferred_element_type=jnp.float32)
    y = nf[...] + s
    mu = jnp.mean(y, axis=-1, keepdims=True)
    var = jnp.mean((y - mu) ** 2, axis=-1, keepdims=True)
    n1 = lg[...] * (y - mu) / jnp.sqrt(var + 1e-5) + lb[...]
    n1_o[...] = n1
    kn_o[...] = jnp.dot(n1, wvnk[...], preferred_element_type=jnp.float32)
    vv_o[...] = jnp.dot(n1, wvnv[...], preferred_element_type=jnp.float32)


def _vn_body(vn, pooled, wq_, wo_, bo_, wn_, bn_, qv_o, vnn_o, vnnt_o):
    v = vn[...]
    qv_o[...] = jnp.dot(v, wq_[...], preferred_element_type=jnp.float32)
    vnew = v + jnp.dot(pooled[...], wo_[...],
                       preferred_element_type=jnp.float32) + bo_[...]
    vnn_o[...] = vnew
    vnnt_o[...] = jnp.dot(vnew, wn_[...],
                          preferred_element_type=jnp.float32) + bn_[...]


def _final_body(n1, vt, rt, sc, w1, b1, w2, b2, w3, b3, tg, tb, wbb, bbb,
                wsc, bsc, wen, ben, n3_o, rto_o, side_o, hn_o):
    node2 = n1[...] + vt[...]
    h = jnp.maximum(jnp.dot(node2, w1[...],
                            preferred_element_type=jnp.float32) + b1[...], 0.0)
    h = jnp.maximum(jnp.dot(h, w2[...],
                            preferred_element_type=jnp.float32) + b2[...], 0.0)
    h = jnp.dot(h, w3[...], preferred_element_type=jnp.float32) + b3[...]
    y = node2 + h
    mu = jnp.mean(y, axis=-1, keepdims=True)
    var = jnp.mean((y - mu) ** 2, axis=-1, keepdims=True)
    n3 = tg[...] * (y - mu) / jnp.sqrt(var + 1e-5) + tb[...]
    n3_o[...] = n3
    r = rt[...]
    nm = r[:, 12:13]
    u = nm * (jnp.dot(n3, wbb[...], preferred_element_type=jnp.float32)
              + bbb[...])
    qx, qy, qz = u[:, 0:1], u[:, 1:2], u[:, 2:3]
    inv = 1.0 / jnp.sqrt(1.0 + qx * qx + qy * qy + qz * qz + 1e-12)
    w_, x_, y_, z_ = inv, qx * inv, qy * inv, qz * inv
    Ru = [1 - 2 * (y_ * y_ + z_ * z_), 2 * (x_ * y_ - z_ * w_), 2 * (x_ * z_ + y_ * w_),
          2 * (x_ * y_ + z_ * w_), 1 - 2 * (x_ * x_ + z_ * z_), 2 * (y_ * z_ - x_ * w_),
          2 * (x_ * z_ - y_ * w_), 2 * (y_ * z_ + x_ * w_), 1 - 2 * (x_ * x_ + y_ * y_)]
    pieces = []
    for i in range(3):
        for k in range(3):
            acc = (r[:, 3 * i:3 * i + 1] * Ru[0 + k] + r[:, 3 * i + 1:3 * i + 2] * Ru[3 + k]
                   + r[:, 3 * i + 2:3 * i + 3] * Ru[6 + k])
            pieces.append(acc)
    for i in range(3):
        acc = (r[:, 9 + i:10 + i] + r[:, 3 * i:3 * i + 1] * u[:, 3:4]
               + r[:, 3 * i + 1:3 * i + 2] * u[:, 4:5]
               + r[:, 3 * i + 2:3 * i + 3] * u[:, 5:6])
        pieces.append(acc)
    B = n3.shape[0]
    lane = jax.lax.broadcasted_iota(jnp.int32, (B, 16), 1)
    rto = jnp.zeros((B, 16), jnp.float32)
    for c, piece in enumerate(pieces):
        rto = rto + jnp.where(lane == c, piece, 0.0)
    rto_o[...] = rto
    side_o[...] = sc[...] + nm * (
        jnp.dot(n3, wsc[...], preferred_element_type=jnp.float32) + bsc[...])
    hn_o[...] = jnp.dot(n3, wen[...], preferred_element_type=jnp.float32) + ben[...]


def _etrans_body(hs, hd, z, w1s, w1d, w1z, b1, w2, b2, eg, eb, e_o):
    t = (jnp.dot(hs[...], w1s[...], preferred_element_type=jnp.float32)
         + jnp.dot(hd[...], w1d[...], preferred_element_type=jnp.float32)
         + jnp.dot(z[...], w1z[...], preferred_element_type=jnp.float32)
         + b1[...])
    t = jnp.maximum(t, 0.0)
    e = jnp.dot(t, w2[...], preferred_element_type=jnp.float32) + b2[...]
    mu = jnp.mean(e, axis=-1, keepdims=True)
    var = jnp.mean((e - mu) ** 2, axis=-1, keepdims=True)
    e_o[...] = eg[...] * (e - mu) / jnp.sqrt(var + 1e-5) + eb[...]


def _call(body, grid, in_specs, out_specs, out_shapes, args):
    return pl.pallas_call(
        body, grid=grid, in_specs=in_specs, out_specs=out_specs,
        out_shape=out_shapes)(*args)


@jax.jit
def kernel(node_features, vn_features, rigids_rot, rigids_trans, sidechain,
           edge_features, params, edge_index, batch, res_mask, noising_mask):
    p = params
    N, C_S = node_features.shape
    E, C_Z = edge_features.shape
    G = vn_features.shape[0]
    C_LAT = sidechain.shape[1]
    src, dst = edge_index[0], edge_index[1]
    f32 = jnp.float32

    def pts_perm(npts_in, sel):
        idx = []
        for j in range(3):
            for h in range(_H):
                for q in sel:
                    idx.append(h * npts_in * 3 + q * 3 + j)
        return np.array(idx, np.int32)

    qp_perm = pts_perm(_PQK, range(_PQK))
    kp_perm = pts_perm(_PQK + _PV, range(_PQK))
    vp_perm = pts_perm(_PQK + _PV, range(_PQK, _PQK + _PV))
    wqp = p['qp_W'][:, qp_perm]
    bqp = p['qp_b'][qp_perm][None]
    wkp = p['kvp_W'][:, kp_perm]
    bkp = p['kvp_b'][kp_perm][None]
    wvp = p['kvp_W'][:, vp_perm]
    bvp = p['kvp_b'][vp_perm][None]
    kcols = np.array([h * 2 * _CH + c for h in range(_H) for c in range(_CH)])
    vcols = kcols + _CH
    wk = p['kv_W'][:, kcols]
    bk = p['kv_b'][kcols][None]
    wv = p['kv_W'][:, vcols]
    bv = p['kv_b'][vcols][None]
    npv = _H * _PV
    Wo = p['out_W'][:C_S]
    pt_rows = lambda i: np.array(
        [C_S + h * _PV * 3 + q * 3 + i for h in range(_H) for q in range(_PV)])
    Wptx = p['out_W'][pt_rows(0)]
    Wpty = p['out_W'][pt_rows(1)]
    Wptz = p['out_W'][pt_rows(2)]
    Wnorm = p['out_W'][C_S + 3 * npv: C_S + 4 * npv]
    Wpair = (p['out_W'][C_S + 4 * npv:]
             .reshape(_H, C_Z, C_S).transpose(1, 0, 2).reshape(C_Z, _H * C_S))

    rt = jnp.concatenate(
        [rigids_rot.reshape(N, 9), rigids_trans,
         noising_mask[:, None].astype(f32), jnp.zeros((N, 3), f32)], axis=1)

    bn = _blk(N, 1000)
    be = _blk(E, 1600)
    bec = _blk(E, 800)

    qc, kc, vc = _call(
        _proj_body, (N // bn,),
        [_rows(bn, C_S), _rows(bn, C_LAT), _rows(bn, 16),
         _full((C_S, C_S)), _full((C_LAT, C_S)), _full((1, C_S)),
         _full((C_S, C_S)), _full((1, C_S)), _full((C_S, C_S)), _full((1, C_S)),
         _full((C_S, C_S)), _full((1, C_S)),
         _full((C_S, 96)), _full((1, 96)), _full((C_S, 96)), _full((1, 96)),
         _full((C_S, 192)), _full((1, 192))],
        [_rows(bn, 224), _rows(bn, 224), _rows(bn, 320)],
        [jax.ShapeDtypeStruct((N, 224), f32),
         jax.ShapeDtypeStruct((N, 224), f32),
         jax.ShapeDtypeStruct((N, 320), f32)],
        (node_features, sidechain, rt,
         p['fuse_W'][:C_S], p['fuse_W'][C_S:], p['fuse_b'][None],
         p['q_W'], p['q_b'][None], wk, bk, wv, bv,
         wqp, bqp, wkp, bkp, wvp, bvp))

    gamma = jax.nn.softplus(p['head_w'])[None]

    def onehot(rows, f):
        mat = np.zeros((rows, _H), np.float32)
        for r_ in range(rows):
            mat[r_, f(r_)] = 1.0
        return jnp.asarray(mat)

    s16 = onehot(C_S, lambda r_: r_ // _CH)
    s4 = onehot(_H * _PQK, lambda r_: r_ // _PQK)
    e16 = onehot(C_S, lambda r_: r_ // _CH).T
    e192 = onehot(192, lambda r_: (r_ % 64) // _PV).T

    (a,) = _call(
        _logit_body, (E // be,),
        [_rows(be, 224), _rows(be, 224),
         _rows(be, C_Z), _full((C_Z, _H)), _full((1, _H)), _full((1, _H)),
         _full((C_S, _H)), _full((_H * _PQK, _H))],
        [_rows(be, _H)],
        [jax.ShapeDtypeStruct((E, _H), f32)],
        (jnp.take(qc, dst, axis=0), jnp.take(kc, src, axis=0),
         edge_features, p['bz_W'], p['bz_b'][None], gamma, s16, s4))

    m = jax.ops.segment_max(a, dst, num_segments=N)
    m = jnp.where(jnp.isfinite(m), m, 0.0)
    ex0 = jnp.exp(a - m[dst])
    den = jax.ops.segment_sum(ex0, dst, num_segments=N)
    md = jnp.concatenate([m, den], axis=1)

    (msg,) = _call(
        _msg_body, (E // bec,),
        [_rows(bec, _H), _rows(bec, 2 * _H), _rows(bec, 320), _rows(bec, C_Z),
         _full((C_S, C_S)), _full((C_Z, _H * C_S)), _full((_H, C_S)),
         _full((_H, 192))],
        [_rows(bec, 320)],
        [jax.ShapeDtypeStruct((E, 320), f32)],
        (a, jnp.take(md, dst, axis=0), jnp.take(vc, src, axis=0),
         edge_features, Wo, Wpair, e16, e192))

    agg = jax.ops.segment_sum(msg, dst, num_segments=N)

    node1, kn, vv = _call(
        _node1_body, (N // bn,),
        [_rows(bn, C_S), _rows(bn, 320), _rows(bn, 16),
         _full((npv, C_S)), _full((npv, C_S)), _full((npv, C_S)),
         _full((npv, C_S)), _full((1, C_S)), _full((1, C_S)), _full((1, C_S)),
         _full((C_S, C_S)), _full((C_S, C_S))],
        [_rows(bn, C_S)] * 3,
        [jax.ShapeDtypeStruct((N, C_S), f32)] * 3,
        (node_features, agg, rt, Wptx, Wpty, Wptz, Wnorm,
         p['out_b'][None], p['ln1_g'][None], p['ln1_b'][None],
         p['vnk_W'], p['vnv_W']))

    Gp = ((G + 7) // 8) * 8
    vn_pad = jnp.pad(vn_features, ((0, Gp - G), (0, 0)))
    dh = C_S // _H

    qv_p, _, _ = _call(
        _vn_body, (1,),
        [_full((Gp, C_S)), _full((Gp, C_S))] + [_full((C_S, C_S)),
         _full((C_S, C_S)), _full((1, C_S)), _full((C_S, C_S)), _full((1, C_S))],
        [_full((Gp, C_S))] * 3,
        [jax.ShapeDtypeStruct((Gp, C_S), f32)] * 3,
        (vn_pad, jnp.zeros((Gp, C_S), f32), p['vnq_W'], p['vno_W'],
         p['vno_b'][None], p['vnn_W'], p['vnn_b'][None]))
    qv = qv_p[:G]

    lg = jnp.sum((qv[batch] * kn).reshape(N, _H, dh), -1) / math.sqrt(dh)
    m2 = jax.ops.segment_max(lg, batch, num_segments=G)
    m2 = jnp.where(jnp.isfinite(m2), m2, 0.0)
    ex2 = jnp.exp(lg - m2[batch])
    den2 = jax.ops.segment_sum(ex2, batch, num_segments=G)
    at2 = ex2 / (den2[batch] + 1e-9)
    wvv = jnp.repeat(at2, dh, axis=1) * vv
    pooled = jax.ops.segment_sum(wvv, batch, num_segments=G)

    _, vn_new_p, vnnt_p = _call(
        _vn_body, (1,),
        [_full((Gp, C_S)), _full((Gp, C_S))] + [_full((C_S, C_S)),
         _full((C_S, C_S)), _full((1, C_S)), _full((C_S, C_S)), _full((1, C_S))],
        [_full((Gp, C_S))] * 3,
        [jax.ShapeDtypeStruct((Gp, C_S), f32)] * 3,
        (vn_pad, jnp.pad(pooled, ((0, Gp - G), (0, 0))), p['vnq_W'],
         p['vno_W'], p['vno_b'][None], p['vnn_W'], p['vnn_b'][None]))
    vn_new = vn_new_p[:G]
    vt = vnnt_p[:G][batch]

    bb_Wp = jnp.pad(p['bb_W'], ((0, 0), (0, 2)))
    bb_bp = jnp.pad(p['bb_b'], (0, 2))[None]
    node3, rto, side_new, hn = _call(
        _final_body, (N // bn,),
        [_rows(bn, C_S), _rows(bn, C_S), _rows(bn, 16), _rows(bn, C_LAT),
         _full((C_S, C_S)), _full((1, C_S)), _full((C_S, C_S)), _full((1, C_S)),
         _full((C_S, C_S)), _full((1, C_S)), _full((1, C_S)), _full((1, C_S)),
         _full((C_S, 8)), _full((1, 8)), _full((C_S, C_LAT)), _full((1, C_LAT)),
         _full((C_S, 64)), _full((1, 64))],
        [_rows(bn, C_S), _rows(bn, 16), _rows(bn, C_LAT), _rows(bn, 64)],
        [jax.ShapeDtypeStruct((N, C_S), f32),
         jax.ShapeDtypeStruct((N, 16), f32),
         jax.ShapeDtypeStruct((N, C_LAT), f32),
         jax.ShapeDtypeStruct((N, 64), f32)],
        (node1, vt, rt, sidechain,
         p['t1_W'], p['t1_b'][None], p['t2_W'], p['t2_b'][None],
         p['t3_W'], p['t3_b'][None], p['tln_g'][None], p['tln_b'][None],
         bb_Wp, bb_bp, p['sc_W'], p['sc_b'][None], p['en_W'], p['en_b'][None]))

    new_R = rto[:, :9].reshape(N, 3, 3)
    new_t = rto[:, 9:12]

    (e,) = _call(
        _etrans_body, (E // be,),
        [_rows(be, 64), _rows(be, 64), _rows(be, C_Z),
         _full((64, C_Z)), _full((64, C_Z)), _full((C_Z, C_Z)), _full((1, C_Z)),
         _full((C_Z, C_Z)), _full((1, C_Z)), _full((1, C_Z)), _full((1, C_Z))],
        [_rows(be, C_Z)],
        [jax.ShapeDtypeStruct((E, C_Z), f32)],
        (jnp.take(hn, src, axis=0), jnp.take(hn, dst, axis=0), edge_features,
         p['e1_W'][:64], p['e1_W'][64:128], p['e1_W'][128:], p['e1_b'][None],
         p['e2_W'], p['e2_b'][None], p['eln_g'][None], p['eln_b'][None]))

    return (node3, vn_new, new_R, new_t, side_new, e)
